# baseline (speedup 1.0000x reference)
import functools

import jax
import jax.numpy as jnp
from jax import lax
from jax.experimental import pallas as pl
from jax.experimental.pallas import tpu as pltpu
from jax.experimental.pallas import tpu_sc as plsc

_NUM_CORES = 2
_NUM_SUBCORES = 16
_S_CHUNK = 2
_N_BUF = 5
_LAG = 4


@functools.lru_cache(maxsize=None)
def _build_gather(num_rows: int, dim: int, seq: int, batch: int):
    assert dim % _NUM_CORES == 0
    d_per_core = dim // _NUM_CORES
    assert batch % _NUM_SUBCORES == 0
    b_chunk = batch // _NUM_SUBCORES
    assert seq % _S_CHUNK == 0
    n_chunks = seq // _S_CHUNK
    assert n_chunks % _N_BUF == 0 and n_chunks > _N_BUF
    chunk = _S_CHUNK * b_chunk
    total = seq * b_chunk

    mesh = plsc.VectorSubcoreMesh(core_axis_name="c", subcore_axis_name="s")

    @functools.partial(
        pl.kernel,
        mesh=mesh,
        out_type=jax.ShapeDtypeStruct((seq, dim, batch), jnp.float32),
        scratch_types=(
            [pltpu.VMEM((total,), jnp.int32)]
            + [pltpu.VMEM((chunk,), jnp.float32) for _ in range(_N_BUF)]
            + [pltpu.VMEM_SHARED((num_rows,), jnp.float32)]
            + [pltpu.SemaphoreType.DMA for _ in range(2 * _N_BUF)]
        ),
    )
    def gather_kernel(tab_t, x_t, out, idx_v, *rest):
        bufs = rest[:_N_BUF]
        colbuf = rest[_N_BUF]
        gsems = rest[_N_BUF + 1 : 2 * _N_BUF + 1]
        ssems = rest[2 * _N_BUF + 1 :]
        cid = lax.axis_index("c")
        sid = lax.axis_index("s")
        b0 = sid * b_chunk

        for s in range(seq):
            pltpu.async_copy(
                x_t.at[s, pl.ds(b0, b_chunk)],
                idx_v.at[pl.ds(s * b_chunk, b_chunk)],
                gsems[s % _N_BUF],
            )
        for s in range(seq):
            pltpu.make_async_copy(
                x_t.at[s, pl.ds(b0, b_chunk)],
                idx_v.at[pl.ds(0, b_chunk)],
                gsems[s % _N_BUF],
            ).wait()

        def wait_store_one(h):
            pltpu.make_async_copy(
                bufs[h].at[pl.ds(0, b_chunk)],
                out.at[0, 0, pl.ds(b0, b_chunk)],
                ssems[h],
            ).wait()

        def start_gather(q, h):
            pltpu.async_copy(
                colbuf.at[idx_v.at[pl.ds(q * chunk, chunk)]], bufs[h], gsems[h]
            )

        def wait_gather(h):
            pltpu.make_async_copy(
                colbuf.at[idx_v.at[pl.ds(0, chunk)]], bufs[h], gsems[h]
            ).wait()

        def service(q, d):
            h = q % _N_BUF
            wait_gather(h)
            for s in range(_S_CHUNK):
                pltpu.async_copy(
                    bufs[h].at[pl.ds(s * b_chunk, b_chunk)],
                    out.at[q * _S_CHUNK + s, d, pl.ds(b0, b_chunk)],
                    ssems[h],
                )

        def feat(k, carry):
            d = cid * d_per_core + k
            plsc.subcore_barrier()

            @pl.when(sid == 0)
            def _():
                pltpu.sync_copy(tab_t.at[d], colbuf)

            plsc.subcore_barrier()
            for q in range(n_chunks):
                h = q % _N_BUF
                if q < _N_BUF:
                    @pl.when(k > 0)
                    def _():
                        for _ in range(_S_CHUNK):
                            wait_store_one(h)
                else:
                    for _ in range(_S_CHUNK):
                        wait_store_one(h)
                start_gather(q, h)
                if q >= _LAG:
                    service(q - _LAG, d)
            for q in range(n_chunks - _LAG, n_chunks):
                service(q, d)
            return carry

        lax.fori_loop(0, d_per_core, feat, 0)
        for h in range(_N_BUF):
            for _ in range(_S_CHUNK):
                wait_store_one(h)

    return gather_kernel


def kernel(x, table):
    batch, seq = x.shape
    num_rows, dim = table.shape
    x_t = x.T.astype(jnp.int32)
    tab_t = table.T
    out_t = _build_gather(num_rows, dim, seq, batch)(tab_t, x_t)
    return jnp.transpose(out_t, (2, 0, 1))


# speedup vs baseline: 1.0002x; 1.0002x over previous
import functools

import jax
import jax.numpy as jnp
from jax import lax
from jax.experimental import pallas as pl
from jax.experimental.pallas import tpu as pltpu
from jax.experimental.pallas import tpu_sc as plsc

_NUM_CORES = 2
_NUM_SUBCORES = 16
_S_CHUNK = 2
_N_BUF = 5
_LAG = 3


@functools.lru_cache(maxsize=None)
def _build_gather(num_rows: int, dim: int, seq: int, batch: int):
    assert dim % _NUM_CORES == 0
    d_per_core = dim // _NUM_CORES
    assert batch % _NUM_SUBCORES == 0
    b_chunk = batch // _NUM_SUBCORES
    assert seq % _S_CHUNK == 0
    n_chunks = seq // _S_CHUNK
    assert n_chunks % _N_BUF == 0 and n_chunks > _N_BUF > _LAG
    chunk = _S_CHUNK * b_chunk
    total = seq * b_chunk

    mesh = plsc.VectorSubcoreMesh(core_axis_name="c", subcore_axis_name="s")

    @functools.partial(
        pl.kernel,
        mesh=mesh,
        out_type=jax.ShapeDtypeStruct((seq, dim, batch), jnp.float32),
        scratch_types=(
            [pltpu.VMEM((total,), jnp.int32)]
            + [pltpu.VMEM((chunk,), jnp.float32) for _ in range(_N_BUF)]
            + [pltpu.VMEM_SHARED((num_rows,), jnp.float32)]
            + [pltpu.SemaphoreType.DMA for _ in range(2 * _N_BUF)]
        ),
    )
    def gather_kernel(tab_t, x_t, out, idx_v, *rest):
        bufs = rest[:_N_BUF]
        colbuf = rest[_N_BUF]
        gsems = rest[_N_BUF + 1 : 2 * _N_BUF + 1]
        ssems = rest[2 * _N_BUF + 1 :]
        cid = lax.axis_index("c")
        sid = lax.axis_index("s")
        b0 = sid * b_chunk

        def idx_copy(s):
            return pltpu.make_async_copy(
                x_t.at[s, pl.ds(b0, b_chunk)],
                idx_v.at[pl.ds(s * b_chunk, b_chunk)],
                gsems[s % _N_BUF],
            )

        def gather_copy(q):
            h = q % _N_BUF
            return pltpu.make_async_copy(
                colbuf.at[idx_v.at[pl.ds(q * chunk, chunk)]], bufs[h], gsems[h]
            )

        def store_copy(q, j, d):
            h = q % _N_BUF
            return pltpu.make_async_copy(
                bufs[h].at[pl.ds(j * b_chunk, b_chunk)],
                out.at[q * _S_CHUNK + j, d, pl.ds(b0, b_chunk)],
                ssems[h],
            )

        for s in range(seq):
            idx_copy(s).start()
        for s in range(seq):
            idx_copy(s).wait()

        def service(q, d):
            gather_copy(q).wait()
            for j in range(_S_CHUNK):
                store_copy(q, j, d).start()

        def drain_stores(q, d):
            for j in range(_S_CHUNK):
                store_copy(q, j, d).wait()

        def feat(k, carry):
            d = cid * d_per_core + k
            plsc.subcore_barrier()

            @pl.when(sid == 0)
            def _():
                pltpu.sync_copy(tab_t.at[d], colbuf)

            plsc.subcore_barrier()
            for q in range(n_chunks):
                if q < _N_BUF:
                    @pl.when(k > 0)
                    def _():
                        drain_stores(q + n_chunks - _N_BUF, d - 1)
                else:
                    drain_stores(q - _N_BUF, d)
                gather_copy(q).start()
                if q >= _LAG:
                    service(q - _LAG, d)
            for q in range(n_chunks - _LAG, n_chunks):
                service(q, d)
            return carry

        lax.fori_loop(0, d_per_core, feat, 0)
        d_last = cid * d_per_core + d_per_core - 1
        for q in range(n_chunks - _N_BUF, n_chunks):
            drain_stores(q, d_last)

    return gather_kernel


def kernel(x, table):
    batch, seq = x.shape
    num_rows, dim = table.shape
    x_t = x.T.astype(jnp.int32)
    tab_t = table.T
    out_t = _build_gather(num_rows, dim, seq, batch)(tab_t, x_t)
    return jnp.transpose(out_t, (2, 0, 1))
